# Initial kernel scaffold; baseline (speedup 1.0000x reference)
import jax
import jax.numpy as jnp
from jax.experimental import pallas as pl


def kernel(weights, t):
    raise NotImplementedError("write your pallas kernel here")


# speedup vs baseline: 63.0504x; 63.0504x over previous
import functools

import jax
import jax.numpy as jnp
from jax.experimental import pallas as pl

_INT_MIN = -2147483648


def _ikey_to_f32(ik):
    bits = jnp.where(ik < 0, jnp.int32(_INT_MIN) - ik, ik)
    return jax.lax.bitcast_convert_type(bits, jnp.float32)


def _topk_mask_body(w_ref, ks_ref, out_ref, *, n_svals):
    w = w_ref[...]
    ks = ks_ref[...]
    r_rows = w.shape[0]

    cnt0 = jnp.sum((w >= 0.0).astype(jnp.int32), axis=1, keepdims=True)
    acc = jnp.where(cnt0 >= ks,
                    jnp.zeros((r_rows, n_svals), jnp.int32),
                    jnp.full((r_rows, n_svals), _INT_MIN, jnp.int32))

    def bit_body(i, acc):
        bitv = jax.lax.shift_left(jnp.int32(1), jnp.int32(30) - i)
        cand = jnp.bitwise_or(acc, bitv)
        candf = _ikey_to_f32(cand)
        cols = []
        for s in range(n_svals):
            ge = w >= candf[:, s:s + 1]
            cols.append(jnp.sum(ge.astype(jnp.int32), axis=1, keepdims=True))
        cnt = jnp.concatenate(cols, axis=1)
        return jnp.where(cnt >= ks, cand, acc)

    acc = jax.lax.fori_loop(0, 31, bit_body, acc)
    theta = _ikey_to_f32(acc)
    for s in range(n_svals):
        out_ref[s, :, :] = w >= theta[:, s:s + 1]


def _topk_masks(weights, ks_t, n_svals):
    b_rows, n = weights.shape
    r = 8
    grid = (b_rows // r,)
    body = functools.partial(_topk_mask_body, n_svals=n_svals)
    return pl.pallas_call(
        body,
        grid=grid,
        in_specs=[
            pl.BlockSpec((r, n), lambda g: (g, 0)),
            pl.BlockSpec((r, n_svals), lambda g: (g, 0)),
        ],
        out_specs=pl.BlockSpec((n_svals, r, n), lambda g: (0, g, 0)),
        out_shape=jax.ShapeDtypeStruct((n_svals, b_rows, n), jnp.bool_),
    )(weights, ks_t)


def kernel(weights, t):
    s_steps = t.shape[0]
    n = weights.shape[-1]
    t_sorted = jnp.sort(t, axis=0)
    ks = ((1.0 - jnp.cos(jnp.pi * t_sorted / 2.0)) * n).astype(jnp.int32)
    ws = 0.5 * jnp.pi * jnp.sin(jnp.pi * t_sorted / 2.0)
    ks_t = jnp.transpose(ks[..., 0])
    masks = _topk_masks(weights, ks_t, s_steps)
    return masks, ws
